# baseline (speedup 1.0000x reference)
import jax
import jax.numpy as jnp
from jax.experimental import pallas as pl
from jax.experimental.pallas import tpu as pltpu

_ROWS_X = 10000
_EA_ELEMS = 320000 * 16
_NEA = 4
_EA_CHUNK = _EA_ELEMS // _NEA
_N = 1 + _NEA


def _copy_body(x_hbm, ea_hbm, x_out, ea_out, x_vmem, ea_vmem, *sems):
    load_sems, store_sems = sems[:_N], sems[_N:]
    loads = [pltpu.make_async_copy(x_hbm, x_vmem, load_sems[0])]
    stores = [pltpu.make_async_copy(x_vmem, x_out, store_sems[0])]
    for i in range(_NEA):
        sl = pl.ds(i * _EA_CHUNK, _EA_CHUNK)
        loads.append(pltpu.make_async_copy(
            ea_hbm.at[sl], ea_vmem.at[sl], load_sems[1 + i]))
        stores.append(pltpu.make_async_copy(
            ea_vmem.at[sl], ea_out.at[sl], store_sems[1 + i]))
    for ld in loads:
        ld.start()
    for ld, st in zip(loads, stores):
        ld.wait()
        st.start()
    for st in stores:
        st.wait()


def kernel(x, edge_index, edge_attr):
    del edge_index
    ea_flat = edge_attr.reshape(_EA_ELEMS)
    x_out, ea_out = pl.pallas_call(
        _copy_body,
        in_specs=[
            pl.BlockSpec(memory_space=pl.ANY),
            pl.BlockSpec(memory_space=pl.ANY),
        ],
        out_specs=[
            pl.BlockSpec(memory_space=pl.ANY),
            pl.BlockSpec(memory_space=pl.ANY),
        ],
        out_shape=[
            jax.ShapeDtypeStruct((_ROWS_X, 128), jnp.float32),
            jax.ShapeDtypeStruct((_EA_ELEMS,), jnp.float32),
        ],
        scratch_shapes=(
            [pltpu.VMEM((_ROWS_X, 128), jnp.float32),
             pltpu.VMEM((_EA_ELEMS,), jnp.float32)]
            + [pltpu.SemaphoreType.DMA] * (2 * _N)
        ),
    )(x, ea_flat)
    return (x_out, ea_out.reshape(320000, 16))


# speedup vs baseline: 1.0062x; 1.0062x over previous
import jax
import jax.numpy as jnp
from jax import lax
from jax.experimental import pallas as pl
from jax.experimental.pallas import tpu as pltpu
from jax.experimental.pallas import tpu_sc as plsc

_EA_ROWS = 320000
_EA_CHUNK = 1000

_SC_MESH = plsc.VectorSubcoreMesh(core_axis_name="c", subcore_axis_name="s")
_NW = _SC_MESH.num_cores * _SC_MESH.num_subcores
_ROWS_PER_W = _EA_ROWS // _NW


def _ea_sc_body(ea_hbm, ea_out, rows_v):
    wid = lax.axis_index("s") * _SC_MESH.num_cores + lax.axis_index("c")
    base = wid * _ROWS_PER_W
    for k in range(_ROWS_PER_W // _EA_CHUNK):
        sl = pl.ds(base + k * _EA_CHUNK, _EA_CHUNK)
        pltpu.sync_copy(ea_hbm.at[sl, :], rows_v)
        pltpu.sync_copy(rows_v, ea_out.at[sl, :])


def _x_tc_body(x_hbm, x_out, x_vmem, ld, st):
    lcopy = pltpu.make_async_copy(x_hbm, x_vmem, ld)
    lcopy.start()
    lcopy.wait()
    scopy = pltpu.make_async_copy(x_vmem, x_out, st)
    scopy.start()
    scopy.wait()


def kernel(x, edge_index, edge_attr):
    del edge_index
    ea_out = pl.kernel(
        _ea_sc_body,
        out_type=jax.ShapeDtypeStruct((_EA_ROWS, 16), jnp.float32),
        mesh=_SC_MESH,
        scratch_types=[pltpu.VMEM((_EA_CHUNK, 16), jnp.float32)],
    )(edge_attr)
    x_out = pl.pallas_call(
        _x_tc_body,
        in_specs=[pl.BlockSpec(memory_space=pl.ANY)],
        out_specs=pl.BlockSpec(memory_space=pl.ANY),
        out_shape=jax.ShapeDtypeStruct((10000, 128), jnp.float32),
        scratch_shapes=[
            pltpu.VMEM((10000, 128), jnp.float32),
            pltpu.SemaphoreType.DMA,
            pltpu.SemaphoreType.DMA,
        ],
    )(x)
    return (x_out, ea_out)
